# baseline (speedup 1.0000x reference)
import functools

import jax
import jax.numpy as jnp
import numpy as np
from jax import lax
from jax.experimental import pallas as pl
from jax.experimental.pallas import tpu as pltpu
from jax.experimental.pallas import tpu_sc as plsc

_N = 10000
_E = 320000
_DIN = 128
_H = 64
_DE = 16
_NB = 16
_NG = 32
_L = 4
_NN = 32

_NC = 2
_NS = 16
_NW = _NC * _NS
_EPW = _E // _NW
_CH = 40

_NP = 10240
_RPT = _NP // _NS

_INV = 1.0 / np.sqrt(_NN)

_f32 = jnp.float32


_K = 5
_SUP = _K * _CH
_NSUP = _EPW // _SUP


def _sc_gather_mul_scatter(D):
    mesh = plsc.VectorSubcoreMesh(core_axis_name="c", subcore_axis_name="s")
    grp = D // 16

    def body(table, gate, srcv, dstv, zeros, out, idx_v, dst_v, rows_v,
             gate_v, shared, isem, dsem, gatesem, gsem, ssem):
        c = lax.axis_index("c")
        s = lax.axis_index("s")
        wid = s * _NC + c
        pltpu.sync_copy(zeros.at[pl.ds(s * _RPT, _RPT)],
                        shared.at[pl.ds(s * _RPT, _RPT)])
        plsc.subcore_barrier()
        base0 = wid * _EPW

        def fire_loads(sc, b):
            base = base0 + sc * _SUP
            pltpu.async_copy(srcv.at[pl.ds(base, _SUP)], idx_v.at[b], isem)
            for k in range(_K):
                pltpu.async_copy(dstv.at[pl.ds(base + k * _CH, _CH)],
                                 dst_v.at[b, k], dsem)
            pltpu.async_copy(gate.at[pl.ds(base, _SUP)], gate_v.at[b],
                             gatesem)

        def wait_idx(b):
            pltpu.make_async_copy(srcv.at[pl.ds(0, _SUP)], idx_v.at[b],
                                  isem).wait()

        def wait_dst(b):
            for k in range(_K):
                pltpu.make_async_copy(dstv.at[pl.ds(0, _CH)], dst_v.at[b, k],
                                      dsem).wait()

        def wait_gate(b):
            pltpu.make_async_copy(gate.at[pl.ds(0, _SUP)], gate_v.at[b],
                                  gatesem).wait()

        def fire_gathers(b):
            for k in range(_K):
                pltpu.async_copy(table.at[idx_v.at[b, pl.ds(k * _CH, _CH)]],
                                 rows_v.at[b, pl.ds(k * _CH, _CH)], gsem)

        def wait_gathers(b):
            for k in range(_K):
                pltpu.make_async_copy(
                    table.at[idx_v.at[b, pl.ds(k * _CH, _CH)]],
                    rows_v.at[b, pl.ds(k * _CH, _CH)], gsem).wait()

        def fire_scatters(b):
            for k in range(_K):
                pltpu.async_copy(rows_v.at[b, pl.ds(k * _CH, _CH)],
                                 shared.at[dst_v.at[b, k]], ssem, add=True)

        def wait_scatters(b):
            for k in range(_K):
                pltpu.make_async_copy(rows_v.at[b, pl.ds(k * _CH, _CH)],
                                      shared.at[dst_v.at[b, k]], ssem,
                                      ).wait()

        def mul(b):
            def mulrow(r4, cr):
                r = r4 * 4
                for dr in range(4):
                    for j in range(grp):
                        sl = pl.ds(j * 16, 16)
                        rows_v[b, r + dr, sl] = (rows_v[b, r + dr, sl]
                                                 * gate_v[b, r + dr, sl])
                return cr

            lax.fori_loop(0, _SUP // 4, mulrow, 0)

        def stage(sc, b, fire_next):
            nb = 1 - b
            wait_gate(b)
            wait_gathers(b)
            if fire_next:
                fire_loads(sc + 1, nb)
            mul(b)
            if fire_next:
                wait_idx(nb)
                fire_gathers(nb)
            wait_dst(b)
            fire_scatters(b)
            wait_scatters(b)

        fire_loads(0, 0)
        wait_idx(0)
        fire_gathers(0)

        def pair(t, cr):
            stage(t * 2, 0, True)
            stage(t * 2 + 1, 1, True)
            return cr

        lax.fori_loop(0, _NSUP // 2 - 1, pair, 0)
        stage(_NSUP - 2, 0, True)
        stage(_NSUP - 1, 1, False)

        plsc.subcore_barrier()
        pltpu.sync_copy(shared.at[pl.ds(s * _RPT, _RPT)],
                        out.at[c, pl.ds(s * _RPT, _RPT)])

    return pl.kernel(
        body,
        out_type=jax.ShapeDtypeStruct((_NC, _NP, D), _f32),
        mesh=mesh,
        compiler_params=pltpu.CompilerParams(use_tc_tiling_on_sc=False),
        scratch_types=[
            pltpu.VMEM((2, _SUP), jnp.int32),
            pltpu.VMEM((2, _K, _CH), jnp.int32),
            pltpu.VMEM((2, _SUP, D), _f32),
            pltpu.VMEM((2, _SUP, D), _f32),
            pltpu.VMEM_SHARED((_NP, D), _f32),
            pltpu.SemaphoreType.DMA,
            pltpu.SemaphoreType.DMA,
            pltpu.SemaphoreType.DMA,
            pltpu.SemaphoreType.DMA,
            pltpu.SemaphoreType.DMA,
        ],
    )


def _sc_scatter_add(D):
    mesh = plsc.VectorSubcoreMesh(core_axis_name="c", subcore_axis_name="s")

    def body(vals, dstv, zeros, out, dst_v, rows_v, shared, vsem, dsem, ssem):
        c = lax.axis_index("c")
        s = lax.axis_index("s")
        wid = s * _NC + c
        pltpu.sync_copy(zeros.at[pl.ds(s * _RPT, _RPT)],
                        shared.at[pl.ds(s * _RPT, _RPT)])
        plsc.subcore_barrier()
        base0 = wid * _EPW

        def fire_loads(sc, b):
            base = base0 + sc * _SUP
            pltpu.async_copy(vals.at[pl.ds(base, _SUP)], rows_v.at[b], vsem)
            for k in range(_K):
                pltpu.async_copy(dstv.at[pl.ds(base + k * _CH, _CH)],
                                 dst_v.at[b, k], dsem)

        def stage(sc, b, fire_next):
            nb = 1 - b
            pltpu.make_async_copy(vals.at[pl.ds(0, _SUP)], rows_v.at[b],
                                  vsem).wait()
            for k in range(_K):
                pltpu.make_async_copy(dstv.at[pl.ds(0, _CH)], dst_v.at[b, k],
                                      dsem).wait()
            if fire_next:
                fire_loads(sc + 1, nb)
            for k in range(_K):
                pltpu.async_copy(rows_v.at[b, pl.ds(k * _CH, _CH)],
                                 shared.at[dst_v.at[b, k]], ssem, add=True)
            for k in range(_K):
                pltpu.make_async_copy(rows_v.at[b, pl.ds(k * _CH, _CH)],
                                      shared.at[dst_v.at[b, k]], ssem,
                                      ).wait()

        fire_loads(0, 0)

        def pair(t, cr):
            stage(t * 2, 0, True)
            stage(t * 2 + 1, 1, True)
            return cr

        lax.fori_loop(0, _NSUP // 2 - 1, pair, 0)
        stage(_NSUP - 2, 0, True)
        stage(_NSUP - 1, 1, False)

        plsc.subcore_barrier()
        pltpu.sync_copy(shared.at[pl.ds(s * _RPT, _RPT)],
                        out.at[c, pl.ds(s * _RPT, _RPT)])

    return pl.kernel(
        body,
        out_type=jax.ShapeDtypeStruct((_NC, _NP, D), _f32),
        mesh=mesh,
        compiler_params=pltpu.CompilerParams(use_tc_tiling_on_sc=False),
        scratch_types=[
            pltpu.VMEM((2, _K, _CH), jnp.int32),
            pltpu.VMEM((2, _SUP, D), _f32),
            pltpu.VMEM_SHARED((_NP, D), _f32),
            pltpu.SemaphoreType.DMA,
            pltpu.SemaphoreType.DMA,
            pltpu.SemaphoreType.DMA,
        ],
    )


_BE = 512
_BN = 512


def _tc_gates2(elen2, w1cat, w2blk):
    half = _BE // 2

    def body(elen2_ref, w1_ref, w2_ref, g0, g1):
        xp = elen2_ref[...]
        x2 = jnp.concatenate([xp[:, :_NB], xp[:, _NB:]], axis=0)
        y = jnp.maximum(jnp.dot(x2, w1_ref[...],
                                preferred_element_type=_f32), 0.0)
        g = jnp.dot(y, w2_ref[...], preferred_element_type=_f32)
        for i, go in enumerate((g0, g1)):
            gi = g[:, i * _H:(i + 1) * _H]
            go[...] = jnp.concatenate([gi[:half], gi[half:]], axis=1)

    out = tuple(jax.ShapeDtypeStruct((_E // 2, 128), _f32) for _ in range(2))
    return pl.pallas_call(
        body,
        grid=(_E // _BE,),
        in_specs=[
            pl.BlockSpec((half, 2 * _NB), lambda i: (i, 0)),
            pl.BlockSpec((_NB, 2 * _H), lambda i: (0, 0)),
            pl.BlockSpec((2 * _H, 2 * _H), lambda i: (0, 0)),
        ],
        out_specs=tuple(pl.BlockSpec((half, 128), lambda i: (i, 0))
                        for _ in range(2)),
        out_shape=out,
    )(elen2, w1cat, w2blk)


def _tc_pre(ni, na, eap, w0, we, wa):
    def body(ni_ref, na_ref, eap_ref, w0_ref, we_ref, wa_ref, hw_ref, bs_ref):
        hw_ref[...] = jnp.dot(ni_ref[...], w0_ref[...],
                              preferred_element_type=_f32)
        ea = eap_ref[0] + eap_ref[1]
        na = na_ref[...]
        for i in range(_L):
            bs_ref[i] = (jnp.dot(ea, we_ref[i], preferred_element_type=_f32)
                         * _INV
                         + jnp.dot(na, wa_ref[i], preferred_element_type=_f32))

    return pl.pallas_call(
        body,
        grid=(_NP // _BN,),
        in_specs=[
            pl.BlockSpec((_BN, _DIN), lambda i: (i, 0)),
            pl.BlockSpec((_BN, _DE), lambda i: (i, 0)),
            pl.BlockSpec((_NC, _BN, _DE), lambda i: (0, i, 0)),
            pl.BlockSpec((_DIN, _H), lambda i: (0, 0)),
            pl.BlockSpec((_L, _DE, _H), lambda i: (0, 0, 0)),
            pl.BlockSpec((_L, _DE, _H), lambda i: (0, 0, 0)),
        ],
        out_specs=(
            pl.BlockSpec((_BN, _H), lambda i: (i, 0)),
            pl.BlockSpec((_L, _BN, _H), lambda i: (0, i, 0)),
        ),
        out_shape=(
            jax.ShapeDtypeStruct((_NP, _H), _f32),
            jax.ShapeDtypeStruct((_L, _NP, _H), _f32),
        ),
    )(ni, na, eap, w0, we, wa)


def _tc_layer(aggp, bs, wnext, layer):
    dn = wnext.shape[1]
    apply_gelu = layer < _L - 1

    def body(aggp_ref, b_ref, w_ref, out_ref):
        h = (aggp_ref[0] + aggp_ref[1]) * _INV + b_ref[0]
        if apply_gelu:
            h = jax.nn.gelu(h)
        out_ref[...] = jnp.dot(h, w_ref[...], preferred_element_type=_f32)

    return pl.pallas_call(
        body,
        grid=(_NP // _BN,),
        in_specs=[
            pl.BlockSpec((_NC, _BN, _H), lambda i: (0, i, 0)),
            pl.BlockSpec((1, _BN, _H), lambda i, L=layer: (L, i, 0)),
            pl.BlockSpec((_H, dn), lambda i: (0, 0)),
        ],
        out_specs=pl.BlockSpec((_BN, dn), lambda i: (i, 0)),
        out_shape=jax.ShapeDtypeStruct((_NP, dn), _f32),
    )(aggp, bs, wnext)


def _tc_final(xnp, batch_p):
    def body(xnp_ref, batch_ref, out_ref):
        n = pl.program_id(0)
        xn = jnp.sum(xnp_ref[0] + xnp_ref[1], axis=1, keepdims=True)
        gids = lax.broadcasted_iota(jnp.int32, (1, _NG), 1)
        onehot = (batch_ref[...] == gids).astype(_f32)
        part = jnp.sum(onehot * xn, axis=0)[:, None] * (1.0 / _NN)

        @pl.when(n == 0)
        def _():
            out_ref[...] = jnp.zeros_like(out_ref)

        out_ref[...] += part

    return pl.pallas_call(
        body,
        grid=(_NP // _BN,),
        in_specs=[
            pl.BlockSpec((_NC, _BN, _DE), lambda i: (0, i, 0)),
            pl.BlockSpec((_BN, 1), lambda i: (i, 0)),
        ],
        out_specs=pl.BlockSpec((_NG, 1), lambda i: (0, 0)),
        out_shape=jax.ShapeDtypeStruct((_NG, 1), _f32),
    )(xnp, batch_p)


def kernel(node_input, node_attr, edge_src, edge_dst, edge_attr,
           edge_length_embedding, batch, W0, W_rest, We, Wa, Wfc1, Wfc2,
           W_fctp):
    pad = _NP - _N
    ni_p = jnp.pad(node_input, ((0, pad), (0, 0)))
    na_p = jnp.pad(node_attr, ((0, pad), (0, 0)))
    batch_p = jnp.pad(batch, (0, pad), constant_values=_NG).reshape(_NP, 1)
    zeros64 = jnp.zeros((_NP, _H), _f32)
    zeros16 = jnp.zeros((_NP, _DE), _f32)

    elen2 = edge_length_embedding.reshape(_E // 2, 2 * _NB)
    w1c01 = jnp.transpose(Wfc1[:2], (1, 0, 2)).reshape(_NB, 2 * _H)
    w1c23 = jnp.transpose(Wfc1[2:], (1, 0, 2)).reshape(_NB, 2 * _H)
    w2b01 = jax.scipy.linalg.block_diag(Wfc2[0], Wfc2[1])
    w2b23 = jax.scipy.linalg.block_diag(Wfc2[2], Wfc2[3])

    eap = _sc_scatter_add(_DE)(edge_attr, edge_dst, zeros16)
    g01 = _tc_gates2(elen2, w1c01, w2b01)
    hw, bs = _tc_pre(ni_p, na_p, eap, W0, We, Wa)
    g23 = _tc_gates2(elen2, w1c23, w2b23)
    gates = [g.reshape(_E, _H) for g in (g01 + g23)]

    gms64 = _sc_gather_mul_scatter(_H)
    for i in range(_L):
        aggp = gms64(hw, gates[i], edge_src, edge_dst, zeros64)
        wnext = W_rest[i] if i < _L - 1 else W_fctp[:, :, 0]
        hw = _tc_layer(aggp, bs, wnext, i)

    xnp = _sc_gather_mul_scatter(_DE)(hw, edge_attr, edge_src, edge_dst,
                                      zeros16)
    return _tc_final(xnp, batch_p)


# speedup vs baseline: 1.4725x; 1.4725x over previous
import functools

import jax
import jax.numpy as jnp
import numpy as np
from jax import lax
from jax.experimental import pallas as pl
from jax.experimental.pallas import tpu as pltpu
from jax.experimental.pallas import tpu_sc as plsc

_N = 10000
_E = 320000
_DIN = 128
_H = 64
_DE = 16
_NB = 16
_NG = 32
_L = 4
_NN = 32

_NC = 2
_NS = 16
_NW = _NC * _NS
_EPW = _E // _NW
_CH = 40

_NP = 10240
_RPT = _NP // _NS

_INV = 1.0 / np.sqrt(_NN)

_f32 = jnp.float32


_K = 5
_SUP = _K * _CH
_NSUP = _EPW // _SUP


def _sc_gather_mul_scatter(D):
    mesh = plsc.VectorSubcoreMesh(core_axis_name="c", subcore_axis_name="s")
    grp = D // 16

    def body(table, gate, srcv, dstv, zeros, out, idx_v, dst_v, rows_v,
             gate_v, shared, isem, dsem, gatesem, gsem, ssem):
        c = lax.axis_index("c")
        s = lax.axis_index("s")
        wid = s * _NC + c
        pltpu.sync_copy(zeros.at[pl.ds(s * _RPT, _RPT)],
                        shared.at[pl.ds(s * _RPT, _RPT)])
        plsc.subcore_barrier()
        base0 = wid * _EPW

        def fire_loads(sc, b):
            base = base0 + sc * _SUP
            pltpu.async_copy(srcv.at[pl.ds(base, _SUP)], idx_v.at[b], isem)
            for k in range(_K):
                pltpu.async_copy(dstv.at[pl.ds(base + k * _CH, _CH)],
                                 dst_v.at[b, k], dsem)
            pltpu.async_copy(gate.at[pl.ds(base, _SUP)], gate_v.at[b],
                             gatesem)

        def wait_idx(b):
            pltpu.make_async_copy(srcv.at[pl.ds(0, _SUP)], idx_v.at[b],
                                  isem).wait()

        def wait_dst(b):
            for k in range(_K):
                pltpu.make_async_copy(dstv.at[pl.ds(0, _CH)], dst_v.at[b, k],
                                      dsem).wait()

        def wait_gate(b):
            pltpu.make_async_copy(gate.at[pl.ds(0, _SUP)], gate_v.at[b],
                                  gatesem).wait()

        def fire_gathers(b):
            for k in range(_K):
                pltpu.async_copy(table.at[idx_v.at[b, pl.ds(k * _CH, _CH)]],
                                 rows_v.at[b, pl.ds(k * _CH, _CH)], gsem)

        def wait_gathers(b):
            for k in range(_K):
                pltpu.make_async_copy(
                    table.at[idx_v.at[b, pl.ds(k * _CH, _CH)]],
                    rows_v.at[b, pl.ds(k * _CH, _CH)], gsem).wait()

        def fire_scatters(b):
            for k in range(_K):
                pltpu.async_copy(rows_v.at[b, pl.ds(k * _CH, _CH)],
                                 shared.at[dst_v.at[b, k]], ssem, add=True)

        def wait_scatters(b):
            for k in range(_K):
                pltpu.make_async_copy(rows_v.at[b, pl.ds(k * _CH, _CH)],
                                      shared.at[dst_v.at[b, k]], ssem,
                                      ).wait()

        def mul(b):
            def mulrow(r4, cr):
                r = r4 * 4
                for dr in range(4):
                    for j in range(grp):
                        sl = pl.ds(j * 16, 16)
                        rows_v[b, r + dr, sl] = (rows_v[b, r + dr, sl]
                                                 * gate_v[b, r + dr, sl])
                return cr

            lax.fori_loop(0, _SUP // 4, mulrow, 0)

        def stage(sc, b, fire_next):
            nb = 1 - b
            wait_gate(b)
            wait_gathers(b)
            if fire_next:
                fire_loads(sc + 1, nb)
            mul(b)
            if fire_next:
                wait_idx(nb)
                fire_gathers(nb)
            wait_dst(b)
            fire_scatters(b)
            wait_scatters(b)

        fire_loads(0, 0)
        wait_idx(0)
        fire_gathers(0)

        def pair(t, cr):
            stage(t * 2, 0, True)
            stage(t * 2 + 1, 1, True)
            return cr

        lax.fori_loop(0, _NSUP // 2 - 1, pair, 0)
        stage(_NSUP - 2, 0, True)
        stage(_NSUP - 1, 1, False)

        plsc.subcore_barrier()
        pltpu.sync_copy(shared.at[pl.ds(s * _RPT, _RPT)],
                        out.at[c, pl.ds(s * _RPT, _RPT)])

    return pl.kernel(
        body,
        out_type=jax.ShapeDtypeStruct((_NC, _NP, D), _f32),
        mesh=mesh,
        compiler_params=pltpu.CompilerParams(use_tc_tiling_on_sc=False),
        scratch_types=[
            pltpu.VMEM((2, _SUP), jnp.int32),
            pltpu.VMEM((2, _K, _CH), jnp.int32),
            pltpu.VMEM((2, _SUP, D), _f32),
            pltpu.VMEM((2, _SUP, D), _f32),
            pltpu.VMEM_SHARED((_NP, D), _f32),
            pltpu.SemaphoreType.DMA,
            pltpu.SemaphoreType.DMA,
            pltpu.SemaphoreType.DMA,
            pltpu.SemaphoreType.DMA,
            pltpu.SemaphoreType.DMA,
        ],
    )


def _sc_scatter_add(D):
    mesh = plsc.VectorSubcoreMesh(core_axis_name="c", subcore_axis_name="s")

    def body(vals, dstv, zeros, out, dst_v, rows_v, shared, vsem, dsem, ssem):
        c = lax.axis_index("c")
        s = lax.axis_index("s")
        wid = s * _NC + c
        pltpu.sync_copy(zeros.at[pl.ds(s * _RPT, _RPT)],
                        shared.at[pl.ds(s * _RPT, _RPT)])
        plsc.subcore_barrier()
        base0 = wid * _EPW

        def fire_loads(sc, b):
            base = base0 + sc * _SUP
            pltpu.async_copy(vals.at[pl.ds(base, _SUP)], rows_v.at[b], vsem)
            for k in range(_K):
                pltpu.async_copy(dstv.at[pl.ds(base + k * _CH, _CH)],
                                 dst_v.at[b, k], dsem)

        def stage(sc, b, fire_next):
            nb = 1 - b
            pltpu.make_async_copy(vals.at[pl.ds(0, _SUP)], rows_v.at[b],
                                  vsem).wait()
            for k in range(_K):
                pltpu.make_async_copy(dstv.at[pl.ds(0, _CH)], dst_v.at[b, k],
                                      dsem).wait()
            if fire_next:
                fire_loads(sc + 1, nb)
            for k in range(_K):
                pltpu.async_copy(rows_v.at[b, pl.ds(k * _CH, _CH)],
                                 shared.at[dst_v.at[b, k]], ssem, add=True)
            for k in range(_K):
                pltpu.make_async_copy(rows_v.at[b, pl.ds(k * _CH, _CH)],
                                      shared.at[dst_v.at[b, k]], ssem,
                                      ).wait()

        fire_loads(0, 0)

        def pair(t, cr):
            stage(t * 2, 0, True)
            stage(t * 2 + 1, 1, True)
            return cr

        lax.fori_loop(0, _NSUP // 2 - 1, pair, 0)
        stage(_NSUP - 2, 0, True)
        stage(_NSUP - 1, 1, False)

        plsc.subcore_barrier()
        pltpu.sync_copy(shared.at[pl.ds(s * _RPT, _RPT)],
                        out.at[c, pl.ds(s * _RPT, _RPT)])

    return pl.kernel(
        body,
        out_type=jax.ShapeDtypeStruct((_NC, _NP, D), _f32),
        mesh=mesh,
        compiler_params=pltpu.CompilerParams(use_tc_tiling_on_sc=False),
        scratch_types=[
            pltpu.VMEM((2, _K, _CH), jnp.int32),
            pltpu.VMEM((2, _SUP, D), _f32),
            pltpu.VMEM_SHARED((_NP, D), _f32),
            pltpu.SemaphoreType.DMA,
            pltpu.SemaphoreType.DMA,
            pltpu.SemaphoreType.DMA,
        ],
    )


_BE = 512
_BN = 512


_BEG = 2048


def _tc_gates(elen2, w1cat, w2blk):
    half = _BEG // 2

    def body(elen2_ref, w1_ref, w2_ref, g0, g1, g2, g3):
        xp = elen2_ref[...]
        x2 = jnp.concatenate([xp[:, :_NB], xp[:, _NB:]], axis=0)
        y = jnp.maximum(jnp.dot(x2, w1_ref[...],
                                preferred_element_type=_f32), 0.0)
        g = jnp.dot(y, w2_ref[...], preferred_element_type=_f32)
        for i, go in enumerate((g0, g1, g2, g3)):
            gi = g[:, i * _H:(i + 1) * _H]
            go[...] = jnp.concatenate([gi[:half], gi[half:]], axis=1)

    out = tuple(jax.ShapeDtypeStruct((_E // 2, 128), _f32) for _ in range(4))
    return pl.pallas_call(
        body,
        grid=(_E // _BEG,),
        in_specs=[
            pl.BlockSpec((half, 2 * _NB), lambda i: (i, 0)),
            pl.BlockSpec((_NB, _L * _H), lambda i: (0, 0)),
            pl.BlockSpec((_L * _H, _L * _H), lambda i: (0, 0)),
        ],
        out_specs=tuple(pl.BlockSpec((half, 128), lambda i: (i, 0))
                        for _ in range(4)),
        out_shape=out,
    )(elen2, w1cat, w2blk)


def _tc_pre(ni, na, eap, w0, we, wa):
    def body(ni_ref, na_ref, eap_ref, w0_ref, we_ref, wa_ref, hw_ref, bs_ref):
        hw_ref[...] = jnp.dot(ni_ref[...], w0_ref[...],
                              preferred_element_type=_f32)
        ea = eap_ref[0] + eap_ref[1]
        na = na_ref[...]
        for i in range(_L):
            bs_ref[i] = (jnp.dot(ea, we_ref[i], preferred_element_type=_f32)
                         * _INV
                         + jnp.dot(na, wa_ref[i], preferred_element_type=_f32))

    return pl.pallas_call(
        body,
        grid=(_NP // _BN,),
        in_specs=[
            pl.BlockSpec((_BN, _DIN), lambda i: (i, 0)),
            pl.BlockSpec((_BN, _DE), lambda i: (i, 0)),
            pl.BlockSpec((_NC, _BN, _DE), lambda i: (0, i, 0)),
            pl.BlockSpec((_DIN, _H), lambda i: (0, 0)),
            pl.BlockSpec((_L, _DE, _H), lambda i: (0, 0, 0)),
            pl.BlockSpec((_L, _DE, _H), lambda i: (0, 0, 0)),
        ],
        out_specs=(
            pl.BlockSpec((_BN, _H), lambda i: (i, 0)),
            pl.BlockSpec((_L, _BN, _H), lambda i: (0, i, 0)),
        ),
        out_shape=(
            jax.ShapeDtypeStruct((_NP, _H), _f32),
            jax.ShapeDtypeStruct((_L, _NP, _H), _f32),
        ),
    )(ni, na, eap, w0, we, wa)


def _tc_layer(aggp, bs, wnext, layer):
    dn = wnext.shape[1]
    apply_gelu = layer < _L - 1

    def body(aggp_ref, b_ref, w_ref, out_ref):
        h = (aggp_ref[0] + aggp_ref[1]) * _INV + b_ref[0]
        if apply_gelu:
            h = jax.nn.gelu(h)
        out_ref[...] = jnp.dot(h, w_ref[...], preferred_element_type=_f32)

    return pl.pallas_call(
        body,
        grid=(_NP // _BN,),
        in_specs=[
            pl.BlockSpec((_NC, _BN, _H), lambda i: (0, i, 0)),
            pl.BlockSpec((1, _BN, _H), lambda i, L=layer: (L, i, 0)),
            pl.BlockSpec((_H, dn), lambda i: (0, 0)),
        ],
        out_specs=pl.BlockSpec((_BN, dn), lambda i: (i, 0)),
        out_shape=jax.ShapeDtypeStruct((_NP, dn), _f32),
    )(aggp, bs, wnext)


def _tc_final(xnp, batch_p):
    def body(xnp_ref, batch_ref, out_ref):
        n = pl.program_id(0)
        xn = jnp.sum(xnp_ref[0] + xnp_ref[1], axis=1, keepdims=True)
        gids = lax.broadcasted_iota(jnp.int32, (1, _NG), 1)
        onehot = (batch_ref[...] == gids).astype(_f32)
        part = jnp.sum(onehot * xn, axis=0)[:, None] * (1.0 / _NN)

        @pl.when(n == 0)
        def _():
            out_ref[...] = jnp.zeros_like(out_ref)

        out_ref[...] += part

    return pl.pallas_call(
        body,
        grid=(_NP // _BN,),
        in_specs=[
            pl.BlockSpec((_NC, _BN, _DE), lambda i: (0, i, 0)),
            pl.BlockSpec((_BN, 1), lambda i: (i, 0)),
        ],
        out_specs=pl.BlockSpec((_NG, 1), lambda i: (0, 0)),
        out_shape=jax.ShapeDtypeStruct((_NG, 1), _f32),
    )(xnp, batch_p)


def kernel(node_input, node_attr, edge_src, edge_dst, edge_attr,
           edge_length_embedding, batch, W0, W_rest, We, Wa, Wfc1, Wfc2,
           W_fctp):
    pad = _NP - _N
    ni_p = jnp.pad(node_input, ((0, pad), (0, 0)))
    na_p = jnp.pad(node_attr, ((0, pad), (0, 0)))
    batch_p = jnp.pad(batch, (0, pad), constant_values=_NG).reshape(_NP, 1)
    zeros64 = jnp.zeros((_NP, _H), _f32)
    zeros16 = jnp.zeros((_NP, _DE), _f32)

    elen2 = edge_length_embedding.reshape(_E // 2, 2 * _NB)
    w1cat = jnp.transpose(Wfc1, (1, 0, 2)).reshape(_NB, _L * _H)
    w2blk = jax.scipy.linalg.block_diag(*[Wfc2[i] for i in range(_L)])

    eap = _sc_scatter_add(_DE)(edge_attr, edge_dst, zeros16)
    gates = [g.reshape(_E, _H) for g in _tc_gates(elen2, w1cat, w2blk)]
    hw, bs = _tc_pre(ni_p, na_p, eap, W0, We, Wa)

    gms64 = _sc_gather_mul_scatter(_H)
    for i in range(_L):
        aggp = gms64(hw, gates[i], edge_src, edge_dst, zeros64)
        wnext = W_rest[i] if i < _L - 1 else W_fctp[:, :, 0]
        hw = _tc_layer(aggp, bs, wnext, i)

    xnp = _sc_gather_mul_scatter(_DE)(hw, edge_attr, edge_src, edge_dst,
                                      zeros16)
    return _tc_final(xnp, batch_p)


# speedup vs baseline: 1.5023x; 1.0202x over previous
import functools

import jax
import jax.numpy as jnp
import numpy as np
from jax import lax
from jax.experimental import pallas as pl
from jax.experimental.pallas import tpu as pltpu
from jax.experimental.pallas import tpu_sc as plsc

_N = 10000
_E = 320000
_DIN = 128
_H = 64
_DE = 16
_NB = 16
_NG = 32
_L = 4
_NN = 32

_NC = 2
_NS = 16
_NW = _NC * _NS
_EPW = _E // _NW
_CH = 40

_NP = 10240
_RPT = _NP // _NS

_INV = 1.0 / np.sqrt(_NN)

_f32 = jnp.float32


_K = 5
_SUP = _K * _CH
_NSUP = _EPW // _SUP


def _sc_gather_mul_scatter(D):
    mesh = plsc.VectorSubcoreMesh(core_axis_name="c", subcore_axis_name="s")
    grp = D // 16

    def body(table, gate, srcv, dstv, zeros, out, idx_v, dst_v, rows_v,
             gate_v, shared, isem, dsem, gatesem, gsem, ssem):
        c = lax.axis_index("c")
        s = lax.axis_index("s")
        wid = s * _NC + c
        pltpu.sync_copy(zeros.at[pl.ds(s * _RPT, _RPT)],
                        shared.at[pl.ds(s * _RPT, _RPT)])
        plsc.subcore_barrier()
        base0 = wid * _EPW

        def fire_loads(sc, b):
            base = base0 + sc * _SUP
            pltpu.async_copy(srcv.at[pl.ds(base, _SUP)], idx_v.at[b], isem)
            for k in range(_K):
                pltpu.async_copy(dstv.at[pl.ds(base + k * _CH, _CH)],
                                 dst_v.at[b, k], dsem)
            pltpu.async_copy(gate.at[pl.ds(base, _SUP)], gate_v.at[b],
                             gatesem)

        def wait_idx(b):
            pltpu.make_async_copy(srcv.at[pl.ds(0, _SUP)], idx_v.at[b],
                                  isem).wait()

        def wait_dst(b):
            for k in range(_K):
                pltpu.make_async_copy(dstv.at[pl.ds(0, _CH)], dst_v.at[b, k],
                                      dsem).wait()

        def wait_gate(b):
            pltpu.make_async_copy(gate.at[pl.ds(0, _SUP)], gate_v.at[b],
                                  gatesem).wait()

        def fire_gathers(b):
            for k in range(_K):
                pltpu.async_copy(table.at[idx_v.at[b, pl.ds(k * _CH, _CH)]],
                                 rows_v.at[b, pl.ds(k * _CH, _CH)], gsem)

        def wait_gathers(b):
            for k in range(_K):
                pltpu.make_async_copy(
                    table.at[idx_v.at[b, pl.ds(k * _CH, _CH)]],
                    rows_v.at[b, pl.ds(k * _CH, _CH)], gsem).wait()

        def fire_scatters(b):
            for k in range(_K):
                pltpu.async_copy(rows_v.at[b, pl.ds(k * _CH, _CH)],
                                 shared.at[dst_v.at[b, k]], ssem, add=True)

        def wait_scatters(b):
            for k in range(_K):
                pltpu.make_async_copy(rows_v.at[b, pl.ds(k * _CH, _CH)],
                                      shared.at[dst_v.at[b, k]], ssem,
                                      ).wait()

        def mul(b):
            def mulrow(r4, cr):
                r = r4 * 4
                for dr in range(4):
                    for j in range(grp):
                        sl = pl.ds(j * 16, 16)
                        rows_v[b, r + dr, sl] = (rows_v[b, r + dr, sl]
                                                 * gate_v[b, r + dr, sl])
                return cr

            lax.fori_loop(0, _SUP // 4, mulrow, 0)

        def stage(sc, b, fire_next):
            nb = 1 - b
            wait_gate(b)
            wait_gathers(b)
            if fire_next:
                fire_loads(sc + 1, nb)
            mul(b)
            if fire_next:
                wait_idx(nb)
                fire_gathers(nb)
            wait_dst(b)
            fire_scatters(b)
            wait_scatters(b)

        fire_loads(0, 0)
        wait_idx(0)
        fire_gathers(0)

        def pair(t, cr):
            stage(t * 2, 0, True)
            stage(t * 2 + 1, 1, True)
            return cr

        lax.fori_loop(0, _NSUP // 2 - 1, pair, 0)
        stage(_NSUP - 2, 0, True)
        stage(_NSUP - 1, 1, False)

        plsc.subcore_barrier()
        pltpu.sync_copy(shared.at[pl.ds(s * _RPT, _RPT)],
                        out.at[c, pl.ds(s * _RPT, _RPT)])

    return pl.kernel(
        body,
        out_type=jax.ShapeDtypeStruct((_NC, _NP, D), _f32),
        mesh=mesh,
        compiler_params=pltpu.CompilerParams(use_tc_tiling_on_sc=False),
        scratch_types=[
            pltpu.VMEM((2, _SUP), jnp.int32),
            pltpu.VMEM((2, _K, _CH), jnp.int32),
            pltpu.VMEM((2, _SUP, D), _f32),
            pltpu.VMEM((2, _SUP, D), _f32),
            pltpu.VMEM_SHARED((_NP, D), _f32),
            pltpu.SemaphoreType.DMA,
            pltpu.SemaphoreType.DMA,
            pltpu.SemaphoreType.DMA,
            pltpu.SemaphoreType.DMA,
            pltpu.SemaphoreType.DMA,
        ],
    )


def _sc_scatter_add(D):
    mesh = plsc.VectorSubcoreMesh(core_axis_name="c", subcore_axis_name="s")

    def body(vals, dstv, zeros, out, dst_v, rows_v, shared, vsem, dsem, ssem):
        c = lax.axis_index("c")
        s = lax.axis_index("s")
        wid = s * _NC + c
        pltpu.sync_copy(zeros.at[pl.ds(s * _RPT, _RPT)],
                        shared.at[pl.ds(s * _RPT, _RPT)])
        plsc.subcore_barrier()
        base0 = wid * _EPW

        def fire_loads(sc, b):
            base = base0 + sc * _SUP
            pltpu.async_copy(vals.at[pl.ds(base, _SUP)], rows_v.at[b], vsem)
            for k in range(_K):
                pltpu.async_copy(dstv.at[pl.ds(base + k * _CH, _CH)],
                                 dst_v.at[b, k], dsem)

        def stage(sc, b, fire_next):
            nb = 1 - b
            pltpu.make_async_copy(vals.at[pl.ds(0, _SUP)], rows_v.at[b],
                                  vsem).wait()
            for k in range(_K):
                pltpu.make_async_copy(dstv.at[pl.ds(0, _CH)], dst_v.at[b, k],
                                      dsem).wait()
            if fire_next:
                fire_loads(sc + 1, nb)
            for k in range(_K):
                pltpu.async_copy(rows_v.at[b, pl.ds(k * _CH, _CH)],
                                 shared.at[dst_v.at[b, k]], ssem, add=True)
            for k in range(_K):
                pltpu.make_async_copy(rows_v.at[b, pl.ds(k * _CH, _CH)],
                                      shared.at[dst_v.at[b, k]], ssem,
                                      ).wait()

        fire_loads(0, 0)

        def pair(t, cr):
            stage(t * 2, 0, True)
            stage(t * 2 + 1, 1, True)
            return cr

        lax.fori_loop(0, _NSUP // 2 - 1, pair, 0)
        stage(_NSUP - 2, 0, True)
        stage(_NSUP - 1, 1, False)

        plsc.subcore_barrier()
        pltpu.sync_copy(shared.at[pl.ds(s * _RPT, _RPT)],
                        out.at[c, pl.ds(s * _RPT, _RPT)])

    return pl.kernel(
        body,
        out_type=jax.ShapeDtypeStruct((_NC, _NP, D), _f32),
        mesh=mesh,
        compiler_params=pltpu.CompilerParams(use_tc_tiling_on_sc=False),
        scratch_types=[
            pltpu.VMEM((2, _K, _CH), jnp.int32),
            pltpu.VMEM((2, _SUP, D), _f32),
            pltpu.VMEM_SHARED((_NP, D), _f32),
            pltpu.SemaphoreType.DMA,
            pltpu.SemaphoreType.DMA,
            pltpu.SemaphoreType.DMA,
        ],
    )


_BE = 512
_BN = 512


_BEG = 2560


def _tc_gates(elen2, w1cat, w2blk):
    half = _BEG // 2

    def body(elen2_ref, w1_ref, w2_ref, g0, g1, g2, g3):
        xp = elen2_ref[...]
        x2 = jnp.concatenate([xp[:, :_NB], xp[:, _NB:]], axis=0)
        y = jnp.maximum(jnp.dot(x2, w1_ref[...],
                                preferred_element_type=_f32), 0.0)
        g = jnp.dot(y, w2_ref[...], preferred_element_type=_f32)
        for i, go in enumerate((g0, g1, g2, g3)):
            gi = g[:, i * _H:(i + 1) * _H]
            go[...] = jnp.concatenate([gi[:half], gi[half:]], axis=1)

    out = tuple(jax.ShapeDtypeStruct((_E // 2, 128), _f32) for _ in range(4))
    return pl.pallas_call(
        body,
        grid=(_E // _BEG,),
        in_specs=[
            pl.BlockSpec((half, 2 * _NB), lambda i: (i, 0)),
            pl.BlockSpec((_NB, _L * _H), lambda i: (0, 0)),
            pl.BlockSpec((_L * _H, _L * _H), lambda i: (0, 0)),
        ],
        out_specs=tuple(pl.BlockSpec((half, 128), lambda i: (i, 0))
                        for _ in range(4)),
        out_shape=out,
    )(elen2, w1cat, w2blk)


def _tc_pre(ni, na, eap, w0, we, wa):
    def body(ni_ref, na_ref, eap_ref, w0_ref, we_ref, wa_ref, hw_ref, bs_ref):
        hw_ref[...] = jnp.dot(ni_ref[...], w0_ref[...],
                              preferred_element_type=_f32)
        ea = eap_ref[0] + eap_ref[1]
        na = na_ref[...]
        for i in range(_L):
            bs_ref[i] = (jnp.dot(ea, we_ref[i], preferred_element_type=_f32)
                         * _INV
                         + jnp.dot(na, wa_ref[i], preferred_element_type=_f32))

    return pl.pallas_call(
        body,
        grid=(_NP // _BN,),
        in_specs=[
            pl.BlockSpec((_BN, _DIN), lambda i: (i, 0)),
            pl.BlockSpec((_BN, _DE), lambda i: (i, 0)),
            pl.BlockSpec((_NC, _BN, _DE), lambda i: (0, i, 0)),
            pl.BlockSpec((_DIN, _H), lambda i: (0, 0)),
            pl.BlockSpec((_L, _DE, _H), lambda i: (0, 0, 0)),
            pl.BlockSpec((_L, _DE, _H), lambda i: (0, 0, 0)),
        ],
        out_specs=(
            pl.BlockSpec((_BN, _H), lambda i: (i, 0)),
            pl.BlockSpec((_L, _BN, _H), lambda i: (0, i, 0)),
        ),
        out_shape=(
            jax.ShapeDtypeStruct((_NP, _H), _f32),
            jax.ShapeDtypeStruct((_L, _NP, _H), _f32),
        ),
    )(ni, na, eap, w0, we, wa)


def _tc_layer(aggp, bs, wnext, layer):
    dn = wnext.shape[1]
    apply_gelu = layer < _L - 1

    def body(aggp_ref, b_ref, w_ref, out_ref):
        h = (aggp_ref[0] + aggp_ref[1]) * _INV + b_ref[0]
        if apply_gelu:
            h = jax.nn.gelu(h)
        out_ref[...] = jnp.dot(h, w_ref[...], preferred_element_type=_f32)

    return pl.pallas_call(
        body,
        grid=(_NP // _BN,),
        in_specs=[
            pl.BlockSpec((_NC, _BN, _H), lambda i: (0, i, 0)),
            pl.BlockSpec((1, _BN, _H), lambda i, L=layer: (L, i, 0)),
            pl.BlockSpec((_H, dn), lambda i: (0, 0)),
        ],
        out_specs=pl.BlockSpec((_BN, dn), lambda i: (i, 0)),
        out_shape=jax.ShapeDtypeStruct((_NP, dn), _f32),
    )(aggp, bs, wnext)


def _tc_final(xnp, batch_p):
    def body(xnp_ref, batch_ref, out_ref):
        n = pl.program_id(0)
        xn = jnp.sum(xnp_ref[0] + xnp_ref[1], axis=1, keepdims=True)
        gids = lax.broadcasted_iota(jnp.int32, (1, _NG), 1)
        onehot = (batch_ref[...] == gids).astype(_f32)
        part = jnp.sum(onehot * xn, axis=0)[:, None] * (1.0 / _NN)

        @pl.when(n == 0)
        def _():
            out_ref[...] = jnp.zeros_like(out_ref)

        out_ref[...] += part

    return pl.pallas_call(
        body,
        grid=(_NP // _BN,),
        in_specs=[
            pl.BlockSpec((_NC, _BN, _DE), lambda i: (0, i, 0)),
            pl.BlockSpec((_BN, 1), lambda i: (i, 0)),
        ],
        out_specs=pl.BlockSpec((_NG, 1), lambda i: (0, 0)),
        out_shape=jax.ShapeDtypeStruct((_NG, 1), _f32),
    )(xnp, batch_p)


def kernel(node_input, node_attr, edge_src, edge_dst, edge_attr,
           edge_length_embedding, batch, W0, W_rest, We, Wa, Wfc1, Wfc2,
           W_fctp):
    pad = _NP - _N
    ni_p = jnp.pad(node_input, ((0, pad), (0, 0)))
    na_p = jnp.pad(node_attr, ((0, pad), (0, 0)))
    batch_p = jnp.pad(batch, (0, pad), constant_values=_NG).reshape(_NP, 1)
    zeros64 = jnp.zeros((_NP, _H), _f32)
    zeros16 = jnp.zeros((_NP, _DE), _f32)

    elen2 = edge_length_embedding.reshape(_E // 2, 2 * _NB)
    w1cat = jnp.transpose(Wfc1, (1, 0, 2)).reshape(_NB, _L * _H)
    w2blk = jax.scipy.linalg.block_diag(*[Wfc2[i] for i in range(_L)])

    eap = _sc_scatter_add(_DE)(edge_attr, edge_dst, zeros16)
    gates = [g.reshape(_E, _H) for g in _tc_gates(elen2, w1cat, w2blk)]
    hw, bs = _tc_pre(ni_p, na_p, eap, W0, We, Wa)

    gms64 = _sc_gather_mul_scatter(_H)
    for i in range(_L):
        aggp = gms64(hw, gates[i], edge_src, edge_dst, zeros64)
        wnext = W_rest[i] if i < _L - 1 else W_fctp[:, :, 0]
        hw = _tc_layer(aggp, bs, wnext, i)

    xnp = _sc_gather_mul_scatter(_DE)(hw, edge_attr, edge_src, edge_dst,
                                      zeros16)
    return _tc_final(xnp, batch_p)
